# baseline (speedup 1.0000x reference)
import functools

import jax
import jax.numpy as jnp
from jax import lax
from jax.experimental import pallas as pl
from jax.experimental.pallas import tpu as pltpu
from jax.experimental.pallas import tpu_sc as plsc

N = 10000
E = 320000
NFEAT = 128
NHID = 256
NCLASS = 64
NLAYERS = 8

NC = 2
NS = 16
LANES = 16
COLS = NCLASS // NC
RPT = N // NS
EPT = E // NS
K = 400
NCHUNK = EPT // K

_ROWBLK = 2000

_DNUMS = lax.GatherDimensionNumbers(
    offset_dims=(), collapsed_slice_dims=(0,), start_index_map=(0,))


def _splat_lane(v, lane):
    idx = jnp.full((LANES, 1), lane, jnp.int32)
    return lax.gather(v, idx, _DNUMS, (1,),
                      mode=lax.GatherScatterMode.PROMISE_IN_BOUNDS)


def _front_body(x_ref, win_ref, bin_ref, wout_ref, o_ref):
    h = jnp.dot(x_ref[...], win_ref[...], preferred_element_type=jnp.float32)
    h = jnp.maximum(h + bin_ref[...], 0.0)
    o_ref[...] = jnp.dot(h, wout_ref[...], preferred_element_type=jnp.float32)


def _front(x, W_in, b_in2, W_out):
    return pl.pallas_call(
        _front_body,
        grid=(N // _ROWBLK,),
        in_specs=[
            pl.BlockSpec((_ROWBLK, NFEAT), lambda i: (i, 0)),
            pl.BlockSpec((NFEAT, NHID), lambda i: (0, 0)),
            pl.BlockSpec((1, NHID), lambda i: (0, 0)),
            pl.BlockSpec((NHID, NCLASS), lambda i: (0, 0)),
        ],
        out_specs=pl.BlockSpec((_ROWBLK, NCLASS), lambda i: (i, 0)),
        out_shape=jax.ShapeDtypeStruct((N, NCLASS), jnp.float32),
    )(x, W_in, b_in2, W_out)


def _out_body(g0_ref, g1_ref, b_ref, o_ref):
    g = jnp.concatenate([g0_ref[...], g1_ref[...]], axis=1) + b_ref[...]
    m = jnp.max(g, axis=1, keepdims=True)
    ex = jnp.exp(g - m)
    s = jnp.sum(ex, axis=1, keepdims=True)
    o_ref[...] = (g - m) - jnp.log(s)


def _outproj(gh0, gh1, b_out2):
    return pl.pallas_call(
        _out_body,
        grid=(N // _ROWBLK,),
        in_specs=[
            pl.BlockSpec((_ROWBLK, COLS), lambda i: (i, 0)),
            pl.BlockSpec((_ROWBLK, COLS), lambda i: (i, 0)),
            pl.BlockSpec((1, NCLASS), lambda i: (0, 0)),
        ],
        out_specs=pl.BlockSpec((_ROWBLK, NCLASS), lambda i: (i, 0)),
        out_shape=jax.ShapeDtypeStruct((N, NCLASS), jnp.float32),
    )(gh0, gh1, b_out2)


def _prop_body(g0f, src1, dst2, wgt1, a1h, a2h,
               gout, ta, tb,
               srcadj_v, dst_v, wv0, wv1,
               rows0_v, rows1_v,
               comb_v, g0r_v, a1_v, a2_v,
               acc, gsem0, gsem1, ssem0, ssem1):
    cid = lax.axis_index("c")
    sid = lax.axis_index("s")
    row0 = sid * RPT
    coff = cid * N
    crow0 = sid * NCHUNK

    zvec = jnp.zeros((LANES,), jnp.float32)

    def zero_comb():
        @plsc.parallel_loop(0, RPT, unroll=8)
        def zrow(r):
            for q in range(COLS // LANES):
                comb_v[r, pl.ds(q * LANES, LANES)] = zvec

    ebase = sid * EPT
    pltpu.sync_copy(src1.at[pl.ds(ebase, EPT)], srcadj_v)
    pltpu.sync_copy(dst2.at[pl.ds(crow0, NCHUNK)], dst_v)

    @plsc.parallel_loop(0, EPT // LANES, unroll=8)
    def adj(r):
        s = pl.ds(r * LANES, LANES)
        srcadj_v[s] = srcadj_v[s] + coff

    zero_comb()
    pltpu.sync_copy(comb_v, acc.at[pl.ds(row0, RPT)])
    pltpu.sync_copy(g0f.at[pl.ds(coff + row0, RPT)], g0r_v)
    plsc.subcore_barrier()

    rows = (rows0_v, rows1_v)
    wv = (wv0, wv1)
    gsem = (gsem0, gsem1)
    ssem = (ssem0, ssem1)
    seq = [g0f, ta, tb, ta, tb, ta, tb, ta, gout]
    for l in range(NLAYERS):
        rd, wr = seq[l], seq[l + 1]
        pltpu.sync_copy(a1h.at[l], a1_v)
        pltpu.sync_copy(a2h.at[l], a2_v)

        pltpu.async_copy(rd.at[srcadj_v.at[pl.ds(0, K)]], rows0_v, gsem0)
        pltpu.async_copy(wgt1.at[pl.ds(ebase, K)], wv0, gsem0)

        def pair(j2, c):
            for b in range(2):
                j = 2 * j2 + b
                rb, ro = rows[b], rows[1 - b]
                wb, wo = wv[b], wv[1 - b]
                gb, go = gsem[b], gsem[1 - b]
                sb, so = ssem[b], ssem[1 - b]
                if b == 0:
                    @pl.when(j2 >= 1)
                    def _():
                        pltpu.make_async_copy(
                            ro, acc.at[dst_v.at[j - 1]], so).wait()
                    pltpu.async_copy(
                        rd.at[srcadj_v.at[pl.ds((j + 1) * K, K)]], ro, go)
                    pltpu.async_copy(
                        wgt1.at[pl.ds(ebase + (j + 1) * K, K)], wo, go)
                else:
                    pltpu.make_async_copy(
                        ro, acc.at[dst_v.at[j - 1]], so).wait()

                    @pl.when(j2 < NCHUNK // 2 - 1)
                    def _():
                        pltpu.async_copy(
                            rd.at[srcadj_v.at[pl.ds((j + 1) * K, K)]], ro, go)
                        pltpu.async_copy(
                            wgt1.at[pl.ds(ebase + (j + 1) * K, K)], wo, go)
                pltpu.make_async_copy(
                    rd.at[srcadj_v.at[pl.ds(j * K, K)]], rb, gb).wait()
                pltpu.make_async_copy(
                    wgt1.at[pl.ds(ebase + j * K, K)], wb, gb).wait()

                @plsc.parallel_loop(0, K // LANES, unroll=4)
                def scale(g):
                    wgrp = wb[pl.ds(g * LANES, LANES)]
                    for es in range(LANES):
                        wsp = _splat_lane(wgrp, es)
                        e = g * LANES + es
                        for q in range(COLS // LANES):
                            s = pl.ds(q * LANES, LANES)
                            rb[e, s] = rb[e, s] * wsp
                pltpu.async_copy(rb, acc.at[dst_v.at[j]], sb, add=True)
            return c

        with jax.named_scope(f"edges{l}"):
            lax.fori_loop(0, NCHUNK // 2, pair, 0)
            pltpu.make_async_copy(
                rows1_v, acc.at[dst_v.at[NCHUNK - 1]], ssem1).wait()
        plsc.subcore_barrier()

        with jax.named_scope(f"comb{l}"):
            a1 = a1_v[...]
            a2 = a2_v[...]
            pltpu.sync_copy(acc.at[pl.ds(row0, RPT)], comb_v)

            @plsc.parallel_loop(0, RPT, unroll=8)
            def crow(r):
                for q in range(COLS // LANES):
                    s = pl.ds(q * LANES, LANES)
                    comb_v[r, s] = a1 * comb_v[r, s] + a2 * g0r_v[r, s]
            pltpu.sync_copy(comb_v, wr.at[pl.ds(coff + row0, RPT)])
            if l < NLAYERS - 1:
                zero_comb()
                pltpu.sync_copy(comb_v, acc.at[pl.ds(row0, RPT)])
            plsc.subcore_barrier()


@functools.partial(jax.jit, static_argnums=())
def _prop(g0f, src1, dst2, wgt1, a1h, a2h):
    mesh = plsc.VectorSubcoreMesh(core_axis_name="c", subcore_axis_name="s")
    f = pl.kernel(
        _prop_body,
        out_type=[jax.ShapeDtypeStruct((NC * N, COLS), jnp.float32)] * 3,
        mesh=mesh,
        scratch_types=[
            pltpu.VMEM((EPT,), jnp.int32),
            pltpu.VMEM((NCHUNK, K), jnp.int32),
            pltpu.VMEM((K,), jnp.float32),
            pltpu.VMEM((K,), jnp.float32),
            pltpu.VMEM((K, COLS), jnp.float32),
            pltpu.VMEM((K, COLS), jnp.float32),
            pltpu.VMEM((RPT, COLS), jnp.float32),
            pltpu.VMEM((RPT, COLS), jnp.float32),
            pltpu.VMEM((LANES,), jnp.float32),
            pltpu.VMEM((LANES,), jnp.float32),
            pltpu.VMEM_SHARED((N, COLS), jnp.float32),
            pltpu.SemaphoreType.DMA,
            pltpu.SemaphoreType.DMA,
            pltpu.SemaphoreType.DMA,
            pltpu.SemaphoreType.DMA,
        ],
        compiler_params=pltpu.CompilerParams(
            use_tc_tiling_on_sc=False, needs_layout_passes=False),
    )
    return f(g0f, src1, dst2, wgt1, a1h, a2h)


def kernel(x, edge_index, edge_weight, W_in, b_in, W_out, b_out, alpha1, alpha2):
    G0 = _front(x, W_in, b_in.reshape(1, NHID), W_out)
    g0f = G0.reshape(N, NC, COLS).transpose(1, 0, 2).reshape(NC * N, COLS)
    src1 = edge_index[1].astype(jnp.int32)
    dst2 = edge_index[0].astype(jnp.int32).reshape(E // K, K)
    a1h = jnp.broadcast_to(alpha1[:, None], (NLAYERS, LANES))
    a2h = jnp.broadcast_to(alpha2[:, None], (NLAYERS, LANES))
    gout, _ta, _tb = _prop(g0f, src1, dst2, edge_weight, a1h, a2h)
    return _outproj(gout[:N], gout[N:], b_out.reshape(1, NCLASS))


# speedup vs baseline: 1.0261x; 1.0261x over previous
import functools

import jax
import jax.numpy as jnp
from jax import lax
from jax.experimental import pallas as pl
from jax.experimental.pallas import tpu as pltpu
from jax.experimental.pallas import tpu_sc as plsc

N = 10000
E = 320000
NFEAT = 128
NHID = 256
NCLASS = 64
NLAYERS = 8

NC = 2
NS = 16
LANES = 16
COLS = NCLASS // NC
RPT = N // NS
EPT = E // NS
K = 400
NCHUNK = EPT // K

_ROWBLK = 2000

_DNUMS = lax.GatherDimensionNumbers(
    offset_dims=(), collapsed_slice_dims=(0,), start_index_map=(0,))


def _splat_lane(v, lane):
    idx = jnp.full((LANES, 1), lane, jnp.int32)
    return lax.gather(v, idx, _DNUMS, (1,),
                      mode=lax.GatherScatterMode.PROMISE_IN_BOUNDS)


def _front_body(x_ref, win_ref, bin_ref, wout_ref, o_ref):
    h = jnp.dot(x_ref[...], win_ref[...], preferred_element_type=jnp.float32)
    h = jnp.maximum(h + bin_ref[...], 0.0)
    for c in range(NC):
        o_ref[c] = jnp.dot(h, wout_ref[:, c * COLS:(c + 1) * COLS],
                           preferred_element_type=jnp.float32)


def _front(x, W_in, b_in2, W_out):
    return pl.pallas_call(
        _front_body,
        grid=(N // _ROWBLK,),
        in_specs=[
            pl.BlockSpec((_ROWBLK, NFEAT), lambda i: (i, 0)),
            pl.BlockSpec((NFEAT, NHID), lambda i: (0, 0)),
            pl.BlockSpec((1, NHID), lambda i: (0, 0)),
            pl.BlockSpec((NHID, NCLASS), lambda i: (0, 0)),
        ],
        out_specs=pl.BlockSpec((NC, _ROWBLK, COLS), lambda i: (0, i, 0)),
        out_shape=jax.ShapeDtypeStruct((NC, N, COLS), jnp.float32),
    )(x, W_in, b_in2, W_out)


def _out_body(g0_ref, g1_ref, b_ref, o_ref):
    g = jnp.concatenate([g0_ref[...], g1_ref[...]], axis=1) + b_ref[...]
    m = jnp.max(g, axis=1, keepdims=True)
    ex = jnp.exp(g - m)
    s = jnp.sum(ex, axis=1, keepdims=True)
    o_ref[...] = (g - m) - jnp.log(s)


def _outproj(gh0, gh1, b_out2):
    return pl.pallas_call(
        _out_body,
        grid=(N // _ROWBLK,),
        in_specs=[
            pl.BlockSpec((_ROWBLK, COLS), lambda i: (i, 0)),
            pl.BlockSpec((_ROWBLK, COLS), lambda i: (i, 0)),
            pl.BlockSpec((1, NCLASS), lambda i: (0, 0)),
        ],
        out_specs=pl.BlockSpec((_ROWBLK, NCLASS), lambda i: (i, 0)),
        out_shape=jax.ShapeDtypeStruct((N, NCLASS), jnp.float32),
    )(gh0, gh1, b_out2)


def _prop_body(g0f, src1, dst2, wgt1, a1h, a2h,
               gout, ta, tb,
               srcadj_v, dst_v, wv0, wv1,
               rows0_v, rows1_v,
               comb_v, g0r_v, a1_v, a2_v,
               acc, gsem0, gsem1, ssem0, ssem1):
    cid = lax.axis_index("c")
    sid = lax.axis_index("s")
    row0 = sid * RPT
    coff = cid * N
    crow0 = sid * NCHUNK

    zvec = jnp.zeros((LANES,), jnp.float32)

    def zero_comb():
        @plsc.parallel_loop(0, RPT, unroll=8)
        def zrow(r):
            for q in range(COLS // LANES):
                comb_v[r, pl.ds(q * LANES, LANES)] = zvec

    ebase = sid * EPT
    pltpu.sync_copy(src1.at[pl.ds(ebase, EPT)], srcadj_v)
    pltpu.sync_copy(dst2.at[pl.ds(crow0, NCHUNK)], dst_v)

    @plsc.parallel_loop(0, EPT // LANES, unroll=8)
    def adj(r):
        s = pl.ds(r * LANES, LANES)
        srcadj_v[s] = srcadj_v[s] + coff

    zero_comb()
    pltpu.sync_copy(comb_v, acc.at[pl.ds(row0, RPT)])
    pltpu.sync_copy(g0f.at[pl.ds(coff + row0, RPT)], g0r_v)
    plsc.subcore_barrier()

    rows = (rows0_v, rows1_v)
    wv = (wv0, wv1)
    gsem = (gsem0, gsem1)
    ssem = (ssem0, ssem1)
    seq = [g0f, ta, tb, ta, tb, ta, tb, ta, gout]
    for l in range(NLAYERS):
        rd, wr = seq[l], seq[l + 1]
        pltpu.sync_copy(a1h.at[l], a1_v)
        pltpu.sync_copy(a2h.at[l], a2_v)

        pltpu.async_copy(rd.at[srcadj_v.at[pl.ds(0, K)]], rows0_v, gsem0)
        pltpu.async_copy(wgt1.at[pl.ds(ebase, K)], wv0, gsem0)

        def pair(j2, c):
            for b in range(2):
                j = 2 * j2 + b
                rb, ro = rows[b], rows[1 - b]
                wb, wo = wv[b], wv[1 - b]
                gb, go = gsem[b], gsem[1 - b]
                sb, so = ssem[b], ssem[1 - b]
                if b == 0:
                    @pl.when(j2 >= 1)
                    def _():
                        pltpu.make_async_copy(
                            ro, acc.at[dst_v.at[j - 1]], so).wait()
                    pltpu.async_copy(
                        rd.at[srcadj_v.at[pl.ds((j + 1) * K, K)]], ro, go)
                    pltpu.async_copy(
                        wgt1.at[pl.ds(ebase + (j + 1) * K, K)], wo, go)
                else:
                    pltpu.make_async_copy(
                        ro, acc.at[dst_v.at[j - 1]], so).wait()

                    @pl.when(j2 < NCHUNK // 2 - 1)
                    def _():
                        pltpu.async_copy(
                            rd.at[srcadj_v.at[pl.ds((j + 1) * K, K)]], ro, go)
                        pltpu.async_copy(
                            wgt1.at[pl.ds(ebase + (j + 1) * K, K)], wo, go)
                pltpu.make_async_copy(
                    rd.at[srcadj_v.at[pl.ds(j * K, K)]], rb, gb).wait()
                pltpu.make_async_copy(
                    wgt1.at[pl.ds(ebase + j * K, K)], wb, gb).wait()

                @plsc.parallel_loop(0, K // LANES, unroll=2)
                def scale(g):
                    wgrp = wb[pl.ds(g * LANES, LANES)]
                    for es in range(LANES):
                        wsp = _splat_lane(wgrp, es)
                        e = g * LANES + es
                        for q in range(COLS // LANES):
                            s = pl.ds(q * LANES, LANES)
                            rb[e, s] = rb[e, s] * wsp
                pltpu.async_copy(rb, acc.at[dst_v.at[j]], sb, add=True)
            return c

        with jax.named_scope(f"edges{l}"):
            lax.fori_loop(0, NCHUNK // 2, pair, 0)
            pltpu.make_async_copy(
                rows1_v, acc.at[dst_v.at[NCHUNK - 1]], ssem1).wait()
        plsc.subcore_barrier()

        with jax.named_scope(f"comb{l}"):
            a1 = a1_v[...]
            a2 = a2_v[...]
            pltpu.sync_copy(acc.at[pl.ds(row0, RPT)], comb_v)

            @plsc.parallel_loop(0, RPT, unroll=8)
            def crow(r):
                for q in range(COLS // LANES):
                    s = pl.ds(q * LANES, LANES)
                    comb_v[r, s] = a1 * comb_v[r, s] + a2 * g0r_v[r, s]
            pltpu.sync_copy(comb_v, wr.at[pl.ds(coff + row0, RPT)])
            if l < NLAYERS - 1:
                zero_comb()
                pltpu.sync_copy(comb_v, acc.at[pl.ds(row0, RPT)])
            plsc.subcore_barrier()


@functools.partial(jax.jit, static_argnums=())
def _prop(g0f, src1, dst2, wgt1, a1h, a2h):
    mesh = plsc.VectorSubcoreMesh(core_axis_name="c", subcore_axis_name="s")
    f = pl.kernel(
        _prop_body,
        out_type=[jax.ShapeDtypeStruct((NC * N, COLS), jnp.float32)] * 3,
        mesh=mesh,
        scratch_types=[
            pltpu.VMEM((EPT,), jnp.int32),
            pltpu.VMEM((NCHUNK, K), jnp.int32),
            pltpu.VMEM((K,), jnp.float32),
            pltpu.VMEM((K,), jnp.float32),
            pltpu.VMEM((K, COLS), jnp.float32),
            pltpu.VMEM((K, COLS), jnp.float32),
            pltpu.VMEM((RPT, COLS), jnp.float32),
            pltpu.VMEM((RPT, COLS), jnp.float32),
            pltpu.VMEM((LANES,), jnp.float32),
            pltpu.VMEM((LANES,), jnp.float32),
            pltpu.VMEM_SHARED((N, COLS), jnp.float32),
            pltpu.SemaphoreType.DMA,
            pltpu.SemaphoreType.DMA,
            pltpu.SemaphoreType.DMA,
            pltpu.SemaphoreType.DMA,
        ],
        compiler_params=pltpu.CompilerParams(
            use_tc_tiling_on_sc=False, needs_layout_passes=False),
    )
    return f(g0f, src1, dst2, wgt1, a1h, a2h)


def kernel(x, edge_index, edge_weight, W_in, b_in, W_out, b_out, alpha1, alpha2):
    G0 = _front(x, W_in, b_in.reshape(1, NHID), W_out)
    g0f = G0.reshape(NC * N, COLS)
    src1 = edge_index[1].astype(jnp.int32)
    dst2 = edge_index[0].astype(jnp.int32).reshape(E // K, K)
    a1h = jnp.broadcast_to(alpha1[:, None], (NLAYERS, LANES))
    a2h = jnp.broadcast_to(alpha2[:, None], (NLAYERS, LANES))
    gout, _ta, _tb = _prop(g0f, src1, dst2, edge_weight, a1h, a2h)
    return _outproj(gout[:N], gout[N:], b_out.reshape(1, NCLASS))


# speedup vs baseline: 1.0455x; 1.0189x over previous
import functools

import jax
import jax.numpy as jnp
from jax import lax
from jax.experimental import pallas as pl
from jax.experimental.pallas import tpu as pltpu
from jax.experimental.pallas import tpu_sc as plsc

N = 10000
E = 320000
NFEAT = 128
NHID = 256
NCLASS = 64
NLAYERS = 8

NC = 2
NS = 16
LANES = 16
COLS = NCLASS // NC
RPT = N // NS
EPT = E // NS
K = 400
NCHUNK = EPT // K

_ROWBLK = 2000

_DNUMS = lax.GatherDimensionNumbers(
    offset_dims=(), collapsed_slice_dims=(0,), start_index_map=(0,))


def _splat_lane(v, lane):
    idx = jnp.full((LANES, 1), lane, jnp.int32)
    return lax.gather(v, idx, _DNUMS, (1,),
                      mode=lax.GatherScatterMode.PROMISE_IN_BOUNDS)


def _front_body(x_ref, win_ref, bin_ref, wout_ref, o_ref):
    h = jnp.dot(x_ref[...], win_ref[...], preferred_element_type=jnp.float32)
    h = jnp.maximum(h + bin_ref[...], 0.0)
    for c in range(NC):
        o_ref[c] = jnp.dot(h, wout_ref[:, c * COLS:(c + 1) * COLS],
                           preferred_element_type=jnp.float32)


def _front(x, W_in, b_in2, W_out):
    return pl.pallas_call(
        _front_body,
        grid=(N // _ROWBLK,),
        in_specs=[
            pl.BlockSpec((_ROWBLK, NFEAT), lambda i: (i, 0)),
            pl.BlockSpec((NFEAT, NHID), lambda i: (0, 0)),
            pl.BlockSpec((1, NHID), lambda i: (0, 0)),
            pl.BlockSpec((NHID, NCLASS), lambda i: (0, 0)),
        ],
        out_specs=pl.BlockSpec((NC, _ROWBLK, COLS), lambda i: (0, i, 0)),
        out_shape=jax.ShapeDtypeStruct((NC, N, COLS), jnp.float32),
    )(x, W_in, b_in2, W_out)


def _out_body(g3_ref, b_ref, o_ref):
    g = jnp.concatenate([g3_ref[0], g3_ref[1]], axis=1) + b_ref[...]
    m = jnp.max(g, axis=1, keepdims=True)
    ex = jnp.exp(g - m)
    s = jnp.sum(ex, axis=1, keepdims=True)
    o_ref[...] = (g - m) - jnp.log(s)


def _outproj(g3, b_out2):
    return pl.pallas_call(
        _out_body,
        grid=(N // _ROWBLK,),
        in_specs=[
            pl.BlockSpec((NC, _ROWBLK, COLS), lambda i: (0, i, 0)),
            pl.BlockSpec((1, NCLASS), lambda i: (0, 0)),
        ],
        out_specs=pl.BlockSpec((_ROWBLK, NCLASS), lambda i: (i, 0)),
        out_shape=jax.ShapeDtypeStruct((N, NCLASS), jnp.float32),
    )(g3, b_out2)


def _prop_body(g0f, src1, dst2, wgt1, a1h, a2h,
               gout, ta, tb,
               srcadj_v, dst_v, wv0, wv1,
               rows0_v, rows1_v,
               comb_v, g0r_v, a1_v, a2_v,
               acc, gsem0, gsem1, ssem0, ssem1):
    cid = lax.axis_index("c")
    sid = lax.axis_index("s")
    row0 = sid * RPT
    coff = cid * N
    crow0 = sid * NCHUNK

    zvec = jnp.zeros((LANES,), jnp.float32)

    def zero_comb():
        @plsc.parallel_loop(0, RPT, unroll=8)
        def zrow(r):
            for q in range(COLS // LANES):
                comb_v[r, pl.ds(q * LANES, LANES)] = zvec

    ebase = sid * EPT
    pltpu.sync_copy(src1.at[pl.ds(ebase, EPT)], srcadj_v)
    pltpu.sync_copy(dst2.at[pl.ds(crow0, NCHUNK)], dst_v)

    @plsc.parallel_loop(0, EPT // LANES, unroll=8)
    def adj(r):
        s = pl.ds(r * LANES, LANES)
        srcadj_v[s] = srcadj_v[s] + coff

    zero_comb()
    pltpu.sync_copy(comb_v, acc.at[pl.ds(row0, RPT)])
    pltpu.sync_copy(g0f.at[pl.ds(coff + row0, RPT)], g0r_v)
    plsc.subcore_barrier()

    rows = (rows0_v, rows1_v)
    wv = (wv0, wv1)
    gsem = (gsem0, gsem1)
    ssem = (ssem0, ssem1)
    seq = [g0f, ta, tb, ta, tb, ta, tb, ta, gout]
    for l in range(NLAYERS):
        rd, wr = seq[l], seq[l + 1]
        pltpu.sync_copy(a1h.at[l], a1_v)
        pltpu.sync_copy(a2h.at[l], a2_v)

        pltpu.async_copy(rd.at[srcadj_v.at[pl.ds(0, K)]], rows0_v, gsem0)
        pltpu.async_copy(wgt1.at[pl.ds(ebase, K)], wv0, gsem0)

        def pair(j2, c):
            for b in range(2):
                j = 2 * j2 + b
                rb, ro = rows[b], rows[1 - b]
                wb, wo = wv[b], wv[1 - b]
                gb, go = gsem[b], gsem[1 - b]
                sb, so = ssem[b], ssem[1 - b]
                if b == 0:
                    @pl.when(j2 >= 1)
                    def _():
                        pltpu.make_async_copy(
                            ro, acc.at[dst_v.at[j - 1]], so).wait()
                    pltpu.async_copy(
                        rd.at[srcadj_v.at[pl.ds((j + 1) * K, K)]], ro, go)
                    pltpu.async_copy(
                        wgt1.at[pl.ds(ebase + (j + 1) * K, K)], wo, go)
                else:
                    pltpu.make_async_copy(
                        ro, acc.at[dst_v.at[j - 1]], so).wait()

                    @pl.when(j2 < NCHUNK // 2 - 1)
                    def _():
                        pltpu.async_copy(
                            rd.at[srcadj_v.at[pl.ds((j + 1) * K, K)]], ro, go)
                        pltpu.async_copy(
                            wgt1.at[pl.ds(ebase + (j + 1) * K, K)], wo, go)
                pltpu.make_async_copy(
                    rd.at[srcadj_v.at[pl.ds(j * K, K)]], rb, gb).wait()
                pltpu.make_async_copy(
                    wgt1.at[pl.ds(ebase + j * K, K)], wb, gb).wait()

                @plsc.parallel_loop(0, K // LANES, unroll=2)
                def scale(g):
                    wgrp = wb[pl.ds(g * LANES, LANES)]
                    for es in range(LANES):
                        wsp = _splat_lane(wgrp, es)
                        e = g * LANES + es
                        for q in range(COLS // LANES):
                            s = pl.ds(q * LANES, LANES)
                            rb[e, s] = rb[e, s] * wsp
                pltpu.async_copy(rb, acc.at[dst_v.at[j]], sb, add=True)
            return c

        with jax.named_scope(f"edges{l}"):
            lax.fori_loop(0, NCHUNK // 2, pair, 0)
            pltpu.make_async_copy(
                rows1_v, acc.at[dst_v.at[NCHUNK - 1]], ssem1).wait()
        plsc.subcore_barrier()

        with jax.named_scope(f"comb{l}"):
            a1 = a1_v[...]
            a2 = a2_v[...]
            pltpu.sync_copy(acc.at[pl.ds(row0, RPT)], comb_v)

            @plsc.parallel_loop(0, RPT, unroll=8)
            def crow(r):
                for q in range(COLS // LANES):
                    s = pl.ds(q * LANES, LANES)
                    comb_v[r, s] = a1 * comb_v[r, s] + a2 * g0r_v[r, s]
            pltpu.sync_copy(comb_v, wr.at[pl.ds(coff + row0, RPT)])
            if l < NLAYERS - 1:
                zero_comb()
                pltpu.sync_copy(comb_v, acc.at[pl.ds(row0, RPT)])
            plsc.subcore_barrier()


@functools.partial(jax.jit, static_argnums=())
def _prop(g0f, src1, dst2, wgt1, a1h, a2h):
    mesh = plsc.VectorSubcoreMesh(core_axis_name="c", subcore_axis_name="s")
    f = pl.kernel(
        _prop_body,
        out_type=[jax.ShapeDtypeStruct((NC * N, COLS), jnp.float32)] * 3,
        mesh=mesh,
        scratch_types=[
            pltpu.VMEM((EPT,), jnp.int32),
            pltpu.VMEM((NCHUNK, K), jnp.int32),
            pltpu.VMEM((K,), jnp.float32),
            pltpu.VMEM((K,), jnp.float32),
            pltpu.VMEM((K, COLS), jnp.float32),
            pltpu.VMEM((K, COLS), jnp.float32),
            pltpu.VMEM((RPT, COLS), jnp.float32),
            pltpu.VMEM((RPT, COLS), jnp.float32),
            pltpu.VMEM((LANES,), jnp.float32),
            pltpu.VMEM((LANES,), jnp.float32),
            pltpu.VMEM_SHARED((N, COLS), jnp.float32),
            pltpu.SemaphoreType.DMA,
            pltpu.SemaphoreType.DMA,
            pltpu.SemaphoreType.DMA,
            pltpu.SemaphoreType.DMA,
        ],
        compiler_params=pltpu.CompilerParams(
            use_tc_tiling_on_sc=False, needs_layout_passes=False),
    )
    return f(g0f, src1, dst2, wgt1, a1h, a2h)


def kernel(x, edge_index, edge_weight, W_in, b_in, W_out, b_out, alpha1, alpha2):
    G0 = _front(x, W_in, b_in.reshape(1, NHID), W_out)
    g0f = G0.reshape(NC * N, COLS)
    src1 = edge_index[1].astype(jnp.int32)
    dst2 = edge_index[0].astype(jnp.int32).reshape(E // K, K)
    a1h = jnp.broadcast_to(alpha1[:, None], (NLAYERS, LANES))
    a2h = jnp.broadcast_to(alpha2[:, None], (NLAYERS, LANES))
    gout, _ta, _tb = _prop(g0f, src1, dst2, edge_weight, a1h, a2h)
    return _outproj(gout.reshape(NC, N, COLS), b_out.reshape(1, NCLASS))


# speedup vs baseline: 1.0547x; 1.0088x over previous
import functools

import jax
import jax.numpy as jnp
from jax import lax
from jax.experimental import pallas as pl
from jax.experimental.pallas import tpu as pltpu
from jax.experimental.pallas import tpu_sc as plsc

N = 10000
E = 320000
NFEAT = 128
NHID = 256
NCLASS = 64
NLAYERS = 8

NC = 2
NS = 16
LANES = 16
COLS = NCLASS // NC
RPT = N // NS
EPT = E // NS
K = 400
NCHUNK = EPT // K

_ROWBLK = 2000

_DNUMS = lax.GatherDimensionNumbers(
    offset_dims=(), collapsed_slice_dims=(0,), start_index_map=(0,))


def _splat_lane(v, lane):
    idx = jnp.full((LANES, 1), lane, jnp.int32)
    return lax.gather(v, idx, _DNUMS, (1,),
                      mode=lax.GatherScatterMode.PROMISE_IN_BOUNDS)


def _front_body(x_ref, win_ref, bin_ref, wout_ref, o_ref):
    h = jnp.dot(x_ref[...], win_ref[...], preferred_element_type=jnp.float32)
    h = jnp.maximum(h + bin_ref[...], 0.0)
    for c in range(NC):
        o_ref[c] = jnp.dot(h, wout_ref[:, c * COLS:(c + 1) * COLS],
                           preferred_element_type=jnp.float32)


def _front(x, W_in, b_in2, W_out):
    return pl.pallas_call(
        _front_body,
        grid=(N // _ROWBLK,),
        in_specs=[
            pl.BlockSpec((_ROWBLK, NFEAT), lambda i: (i, 0)),
            pl.BlockSpec((NFEAT, NHID), lambda i: (0, 0)),
            pl.BlockSpec((1, NHID), lambda i: (0, 0)),
            pl.BlockSpec((NHID, NCLASS), lambda i: (0, 0)),
        ],
        out_specs=pl.BlockSpec((NC, _ROWBLK, COLS), lambda i: (0, i, 0)),
        out_shape=jax.ShapeDtypeStruct((NC, N, COLS), jnp.float32),
    )(x, W_in, b_in2, W_out)


def _out_body(g3_ref, b_ref, o_ref):
    g = jnp.concatenate([g3_ref[0], g3_ref[1]], axis=1) + b_ref[...]
    m = jnp.max(g, axis=1, keepdims=True)
    ex = jnp.exp(g - m)
    s = jnp.sum(ex, axis=1, keepdims=True)
    o_ref[...] = (g - m) - jnp.log(s)


def _outproj(g3, b_out2):
    return pl.pallas_call(
        _out_body,
        grid=(N // _ROWBLK,),
        in_specs=[
            pl.BlockSpec((NC, _ROWBLK, COLS), lambda i: (0, i, 0)),
            pl.BlockSpec((1, NCLASS), lambda i: (0, 0)),
        ],
        out_specs=pl.BlockSpec((_ROWBLK, NCLASS), lambda i: (i, 0)),
        out_shape=jax.ShapeDtypeStruct((N, NCLASS), jnp.float32),
    )(g3, b_out2)


def _prop_body(g0f, src1, dst2, wgt1, a1h, a2h,
               gout, ta, tb,
               srcadj_v, dst_v, wv0, wv1,
               rows0_v, rows1_v,
               comb_v, g0r_v, a1_v, a2_v,
               acc, gsem0, gsem1, ssem0, ssem1):
    cid = lax.axis_index("c")
    sid = lax.axis_index("s")
    row0 = sid * RPT
    coff = cid * N
    crow0 = sid * NCHUNK

    zvec = jnp.zeros((LANES,), jnp.float32)

    def zero_comb():
        @plsc.parallel_loop(0, RPT, unroll=8)
        def zrow(r):
            for q in range(COLS // LANES):
                comb_v[r, pl.ds(q * LANES, LANES)] = zvec

    ebase = sid * EPT
    pltpu.sync_copy(src1.at[pl.ds(ebase, EPT)], srcadj_v)
    pltpu.sync_copy(dst2.at[pl.ds(crow0, NCHUNK)], dst_v)
    pltpu.sync_copy(a1h, a1_v)
    pltpu.sync_copy(a2h, a2_v)

    @plsc.parallel_loop(0, EPT // LANES, unroll=8)
    def adj(r):
        s = pl.ds(r * LANES, LANES)
        srcadj_v[s] = srcadj_v[s] + coff

    zero_comb()
    pltpu.sync_copy(comb_v, acc.at[pl.ds(row0, RPT)])
    pltpu.sync_copy(g0f.at[pl.ds(coff + row0, RPT)], g0r_v)
    plsc.subcore_barrier()

    rows = (rows0_v, rows1_v)
    wv = (wv0, wv1)
    gsem = (gsem0, gsem1)
    ssem = (ssem0, ssem1)
    seq = [g0f, ta, tb, ta, tb, ta, tb, ta, gout]
    for l in range(NLAYERS):
        rd, wr = seq[l], seq[l + 1]

        pltpu.async_copy(rd.at[srcadj_v.at[pl.ds(0, K)]], rows0_v, gsem0)
        pltpu.async_copy(wgt1.at[pl.ds(ebase, K)], wv0, gsem0)

        def pair(j2, c):
            for b in range(2):
                j = 2 * j2 + b
                rb, ro = rows[b], rows[1 - b]
                wb, wo = wv[b], wv[1 - b]
                gb, go = gsem[b], gsem[1 - b]
                sb, so = ssem[b], ssem[1 - b]
                if b == 0:
                    @pl.when(j2 >= 1)
                    def _():
                        pltpu.make_async_copy(
                            ro, acc.at[dst_v.at[j - 1]], so).wait()
                    pltpu.async_copy(
                        rd.at[srcadj_v.at[pl.ds((j + 1) * K, K)]], ro, go)
                    pltpu.async_copy(
                        wgt1.at[pl.ds(ebase + (j + 1) * K, K)], wo, go)
                else:
                    pltpu.make_async_copy(
                        ro, acc.at[dst_v.at[j - 1]], so).wait()

                    @pl.when(j2 < NCHUNK // 2 - 1)
                    def _():
                        pltpu.async_copy(
                            rd.at[srcadj_v.at[pl.ds((j + 1) * K, K)]], ro, go)
                        pltpu.async_copy(
                            wgt1.at[pl.ds(ebase + (j + 1) * K, K)], wo, go)
                pltpu.make_async_copy(
                    rd.at[srcadj_v.at[pl.ds(j * K, K)]], rb, gb).wait()
                pltpu.make_async_copy(
                    wgt1.at[pl.ds(ebase + j * K, K)], wb, gb).wait()

                @plsc.parallel_loop(0, K // LANES, unroll=2)
                def scale(g):
                    wgrp = wb[pl.ds(g * LANES, LANES)]
                    for es in range(LANES):
                        wsp = _splat_lane(wgrp, es)
                        e = g * LANES + es
                        for q in range(COLS // LANES):
                            s = pl.ds(q * LANES, LANES)
                            rb[e, s] = rb[e, s] * wsp
                pltpu.async_copy(rb, acc.at[dst_v.at[j]], sb, add=True)
            return c

        with jax.named_scope(f"edges{l}"):
            lax.fori_loop(0, NCHUNK // 2, pair, 0)
            pltpu.make_async_copy(
                rows1_v, acc.at[dst_v.at[NCHUNK - 1]], ssem1).wait()
        plsc.subcore_barrier()

        with jax.named_scope(f"comb{l}"):
            a1 = a1_v[l, :]
            a2 = a2_v[l, :]
            pltpu.sync_copy(acc.at[pl.ds(row0, RPT)], comb_v)

            @plsc.parallel_loop(0, RPT, unroll=8)
            def crow(r):
                for q in range(COLS // LANES):
                    s = pl.ds(q * LANES, LANES)
                    comb_v[r, s] = a1 * comb_v[r, s] + a2 * g0r_v[r, s]
            pltpu.sync_copy(comb_v, wr.at[pl.ds(coff + row0, RPT)])
            if l < NLAYERS - 1:
                zero_comb()
                pltpu.sync_copy(comb_v, acc.at[pl.ds(row0, RPT)])
            plsc.subcore_barrier()


@functools.partial(jax.jit, static_argnums=())
def _prop(g0f, src1, dst2, wgt1, a1h, a2h):
    mesh = plsc.VectorSubcoreMesh(core_axis_name="c", subcore_axis_name="s")
    f = pl.kernel(
        _prop_body,
        out_type=[jax.ShapeDtypeStruct((NC * N, COLS), jnp.float32)] * 3,
        mesh=mesh,
        scratch_types=[
            pltpu.VMEM((EPT,), jnp.int32),
            pltpu.VMEM((NCHUNK, K), jnp.int32),
            pltpu.VMEM((K,), jnp.float32),
            pltpu.VMEM((K,), jnp.float32),
            pltpu.VMEM((K, COLS), jnp.float32),
            pltpu.VMEM((K, COLS), jnp.float32),
            pltpu.VMEM((RPT, COLS), jnp.float32),
            pltpu.VMEM((RPT, COLS), jnp.float32),
            pltpu.VMEM((NLAYERS, LANES), jnp.float32),
            pltpu.VMEM((NLAYERS, LANES), jnp.float32),
            pltpu.VMEM_SHARED((N, COLS), jnp.float32),
            pltpu.SemaphoreType.DMA,
            pltpu.SemaphoreType.DMA,
            pltpu.SemaphoreType.DMA,
            pltpu.SemaphoreType.DMA,
        ],
        compiler_params=pltpu.CompilerParams(
            use_tc_tiling_on_sc=False, needs_layout_passes=False),
    )
    return f(g0f, src1, dst2, wgt1, a1h, a2h)


def kernel(x, edge_index, edge_weight, W_in, b_in, W_out, b_out, alpha1, alpha2):
    G0 = _front(x, W_in, b_in.reshape(1, NHID), W_out)
    g0f = G0.reshape(NC * N, COLS)
    src1 = edge_index[1].astype(jnp.int32)
    dst2 = edge_index[0].astype(jnp.int32).reshape(E // K, K)
    a1h = jnp.broadcast_to(alpha1[:, None], (NLAYERS, LANES))
    a2h = jnp.broadcast_to(alpha2[:, None], (NLAYERS, LANES))
    gout, _ta, _tb = _prop(g0f, src1, dst2, edge_weight, a1h, a2h)
    return _outproj(gout.reshape(NC, N, COLS), b_out.reshape(1, NCLASS))
